# baseline (speedup 1.0000x reference)
import jax
import jax.numpy as jnp
from jax import lax
from jax.experimental import pallas as pl
from jax.experimental.pallas import tpu as pltpu
from jax.experimental.pallas import tpu_sc as plsc

_N = 100000
_NPAD = 102400
_CH = 10240
_NCHK = 5
_EDGES = 1600000
_EROWS = 12544
_E2 = _EROWS * 128
_RPT = _EROWS // 16
_BB = 8
_NBAT = _RPT // _BB
_WPT = _CH // 16
_ZR = 640
_NG = 64
_HID = 128
_NCG = _HID // 16
_BLK = 512
_GRID = _NPAD // _BLK


def _sc_mesh():
  return plsc.VectorSubcoreMesh(
      core_axis_name="c", subcore_axis_name="s", num_cores=2, num_subcores=16)


def _make_sc_pass(nplanes, with_gather):
  scratch = [
      pltpu.VMEM((_BB, 128), jnp.int32),
      pltpu.VMEM((_BB, 128), jnp.int32),
      pltpu.VMEM((128, 16), jnp.float32),
      pltpu.VMEM((_ZR, 16), jnp.float32),
      pltpu.VMEM_SHARED((_CH, 16), jnp.float32),
      pltpu.SemaphoreType.DMA,
  ]

  def body(*refs):
    if with_gather:
      srcm, dstm, tbl = refs[0], refs[1], refs[2]
      out = refs[3]
      dstb, srcb, rows, zbuf, acc, sem = refs[4:]
    else:
      srcm, dstm = refs[0], refs[1]
      out = refs[2]
      dstb, srcb, rows, zbuf, acc, sem = refs[3:]

    core = lax.axis_index("c")
    sub = lax.axis_index("s")
    row0 = sub * _RPT

    def zfill(i, _):
      zbuf[i, :] = jnp.zeros((16,), jnp.float32)
      return 0
    lax.fori_loop(0, _ZR, zfill, 0)

    if not with_gather:
      def ofill(i, _):
        rows[i, :] = jnp.ones((16,), jnp.float32)
        return 0
      lax.fori_loop(0, 128, ofill, 0)

    def plane_loop(p, _):
     def chunk_loop(k, _):
      cidx = core * _NCHK + k

      def zcp(i, _):
        pltpu.sync_copy(zbuf, acc.at[pl.ds(sub * _WPT + i * _ZR, _ZR)])
        return 0
      lax.fori_loop(0, _WPT // _ZR, zcp, 0)
      plsc.subcore_barrier()

      def blk(b, _):
        ebase = row0 + b * _BB
        pltpu.sync_copy(dstm.at[cidx, pl.ds(ebase, _BB)], dstb)
        if with_gather:
          pltpu.sync_copy(srcm.at[cidx, pl.ds(ebase, _BB)], srcb)

        def row(t, _):
          if with_gather:
            pltpu.async_copy(
                tbl.at[p].at[plsc.Indices(srcb.at[t], ignored_value=-1)],
                rows, sem).wait()
          pltpu.sync_copy(
              rows, acc.at[plsc.Indices(dstb.at[t], ignored_value=-1)],
              add=True)
          return 0
        lax.fori_loop(0, _BB, row, 0)
        return 0
      lax.fori_loop(0, _NBAT, blk, 0)
      plsc.subcore_barrier()

      def wb(i, _):
        pltpu.sync_copy(
            acc.at[pl.ds(sub * _WPT + i * _ZR, _ZR)],
            out.at[p, pl.ds(cidx * _CH + sub * _WPT + i * _ZR, _ZR)])
        return 0
      lax.fori_loop(0, _WPT // _ZR, wb, 0)
      plsc.subcore_barrier()
      return 0
     lax.fori_loop(0, _NCHK, chunk_loop, 0)
     return 0

    lax.fori_loop(0, nplanes, plane_loop, 0)

  return pl.kernel(
      body,
      out_type=pltpu.HBM((nplanes, _NPAD, 16), jnp.float32),
      mesh=_sc_mesh(),
      scratch_types=scratch,
      compiler_params=pltpu.CompilerParams(use_tc_tiling_on_sc=False),
  )


def _row_spec(shape_prefix):
  nd = len(shape_prefix)
  return pl.BlockSpec(shape_prefix + (_BLK, 16),
                      lambda i: (0,) * nd + (i, 0))


def _full_spec(shape):
  return pl.BlockSpec(shape, lambda i: (0,) * len(shape))


def _t0_body(deg_ref, xp_ref, dinv_ref, xt_ref):
  dinv = lax.rsqrt(1.0 + deg_ref[...])
  dinv_ref[...] = dinv
  xt_ref[...] = dinv * xp_ref[...]


def _t1_body(s1_ref, xt_ref, dinv_ref, w1_ref, b1_ref, w2_ref, out_ref):
  dinvc = dinv_ref[...][:, 0:1]
  agg = dinvc * (s1_ref[...] + xt_ref[...])
  h1 = jnp.maximum(
      jnp.dot(agg, w1_ref[...], preferred_element_type=jnp.float32)
      + b1_ref[...], 0.0)
  u2 = dinvc * jnp.dot(h1, w2_ref[...], preferred_element_type=jnp.float32)
  out_ref[...] = jnp.stack([u2[:, i * 16:(i + 1) * 16] for i in range(_NCG)])


def _t2_body(s_ref, u_ref, dinv_ref, b_ref, w_ref, out_ref):
  s = s_ref[...]
  u = u_ref[...]
  dinvc = dinv_ref[...][:, 0:1]
  b = b_ref[...]
  w = w_ref[...]
  u3 = None
  for p in range(_NCG):
    agg = dinvc * (s[p] + u[p])
    h2 = jnp.maximum(agg + b[:, p * 16:(p + 1) * 16], 0.0)
    part = jnp.dot(h2, w[p * 16:(p + 1) * 16, :],
                   preferred_element_type=jnp.float32)
    u3 = part if u3 is None else u3 + part
  u3 = dinvc * u3
  out_ref[...] = jnp.stack([u3[:, i * 16:(i + 1) * 16] for i in range(_NCG)])


def _t3_body(*refs):
  s_ref, u_ref, dinv_ref, b_ref, batch_ref = refs[:5]
  souts = refs[5:5 + _NCG]
  cnt_ref = refs[5 + _NCG]
  s = s_ref[...]
  u = u_ref[...]
  dinvc = dinv_ref[...][:, 0:1]
  b = b_ref[...]
  bv = batch_ref[...][:, 0]
  ids = lax.broadcasted_iota(jnp.int32, (_BLK, _NG), 1)
  oh = (bv[:, None] == ids).astype(jnp.float32)

  @pl.when(pl.program_id(0) == 0)
  def _init():
    for i in range(_NCG):
      souts[i][...] = jnp.zeros((_NG, 16), jnp.float32)
    cnt_ref[...] = jnp.zeros((_NG, 16), jnp.float32)

  for p in range(_NCG):
    h3 = jnp.maximum(dinvc * (s[p] + u[p])
                     + b[:, p * 16:(p + 1) * 16], 0.0)
    souts[p][...] += lax.dot_general(
        oh, h3, (((0,), (0,)), ((), ())), preferred_element_type=jnp.float32)
  c = jnp.sum(oh, axis=0)
  cnt_ref[...] += jnp.broadcast_to(c[:, None], (_NG, 16))


def _t4_body(*refs):
  sg = refs[0:_NCG]
  cnt_ref, wf1_ref, bf1_ref, wf2_ref, bf2_ref, out_ref = refs[_NCG:]
  inv_cnt = 1.0 / jnp.maximum(cnt_ref[...][:, 0:1], 1.0)
  wf1 = wf1_ref[...]
  acc = None
  for p in range(_NCG):
    g = sg[p][...] * inv_cnt
    part = jnp.dot(g, wf1[p * 16:(p + 1) * 16, :],
                   preferred_element_type=jnp.float32)
    acc = part if acc is None else acc + part
  fc1 = jnp.maximum(acc + bf1_ref[...], 0.0)
  out_ref[...] = (jnp.dot(fc1, wf2_ref[...],
                          preferred_element_type=jnp.float32) + bf2_ref[...])


def kernel(x, edge_index, batch, Wc1, bc1, Wc2, bc2, Wc3, bc3,
           Wf1, bf1, Wf2, bf2):
  f32 = jnp.float32
  npadextra = _NPAD - _N
  epadextra = _E2 - _EDGES

  pad_ids = jnp.arange(epadextra, dtype=jnp.int32)
  src_p = jnp.concatenate([edge_index[0], pad_ids % _N])
  dst_p = jnp.concatenate([edge_index[1], _N + (pad_ids % npadextra)])
  neg1 = jnp.int32(-1)
  dparts, sparts = [], []
  for chunk in range(2 * _NCHK):
    base = chunk * _CH
    inr = (dst_p >= base) & (dst_p < base + _CH)
    dparts.append(jnp.where(inr, dst_p - base, neg1))
    sparts.append(jnp.where(inr, src_p, neg1))
  dstm = jnp.stack(dparts).reshape(2 * _NCHK, _EROWS, 128)
  srcm = jnp.stack(sparts).reshape(2 * _NCHK, _EROWS, 128)

  xpad = jnp.pad(x, ((0, npadextra), (0, 16 - x.shape[1])))
  w1p = jnp.pad(Wc1, ((0, 16 - Wc1.shape[0]), (0, 0)))
  batchcol = jnp.pad(batch, (0, npadextra),
                     constant_values=_NG).reshape(_NPAD, 1)
  wf2p = jnp.pad(Wf2, ((0, 0), (0, 128 - Wf2.shape[1])))
  bf2p = jnp.pad(bf2, (0, 128 - bf2.shape[0])).reshape(1, 128)

  deg = _make_sc_pass(1, False)(srcm, dstm)[0]

  dinv16, xt = pl.pallas_call(
      _t0_body,
      grid=(_GRID,),
      in_specs=[_row_spec(()), _row_spec(())],
      out_specs=[_row_spec(()), _row_spec(())],
      out_shape=[jax.ShapeDtypeStruct((_NPAD, 16), f32)] * 2,
  )(deg, xpad)

  s1 = _make_sc_pass(1, True)(srcm, dstm, xt[None])[0]

  cg_shape = jax.ShapeDtypeStruct((_NCG, _NPAD, 16), f32)
  u2s = pl.pallas_call(
      _t1_body,
      grid=(_GRID,),
      in_specs=[_row_spec(()), _row_spec(()), _row_spec(()),
                _full_spec((16, _HID)), _full_spec((1, _HID)),
                _full_spec((_HID, _HID))],
      out_specs=_row_spec((_NCG,)),
      out_shape=cg_shape,
  )(s1, xt, dinv16, w1p, bc1.reshape(1, _HID), Wc2)

  s2 = _make_sc_pass(_NCG, True)(srcm, dstm, u2s)

  u3s = pl.pallas_call(
      _t2_body,
      grid=(_GRID,),
      in_specs=[_row_spec((_NCG,)), _row_spec((_NCG,)), _row_spec(()),
                _full_spec((1, _HID)), _full_spec((_HID, _HID))],
      out_specs=_row_spec((_NCG,)),
      out_shape=cg_shape,
  )(s2, u2s, dinv16, bc2.reshape(1, _HID), Wc3)

  s3 = _make_sc_pass(_NCG, True)(srcm, dstm, u3s)

  pool = pl.pallas_call(
      _t3_body,
      grid=(_GRID,),
      in_specs=[_row_spec((_NCG,)), _row_spec((_NCG,)), _row_spec(()),
                _full_spec((1, _HID)),
                pl.BlockSpec((_BLK, 1), lambda i: (i, 0))],
      out_specs=[_full_spec((_NG, 16))] * (_NCG + 1),
      out_shape=[jax.ShapeDtypeStruct((_NG, 16), f32)] * (_NCG + 1),
  )(s3, u3s, dinv16, bc3.reshape(1, _HID), batchcol)
  sumsg, cnt = pool[:_NCG], pool[_NCG]

  out128 = pl.pallas_call(
      _t4_body,
      grid=(1,),
      in_specs=[_full_spec((_NG, 16))] * (_NCG + 1)
      + [_full_spec((_HID, _NG)), _full_spec((1, _NG)),
         _full_spec((_NG, 128)), _full_spec((1, 128))],
      out_specs=_full_spec((_NG, 128)),
      out_shape=jax.ShapeDtypeStruct((_NG, 128), f32),
  )(*sumsg, cnt, Wf1, bf1.reshape(1, _NG), wf2p, bf2p)

  return out128[:, :1]


# speedup vs baseline: 1.9775x; 1.9775x over previous
import jax
import jax.numpy as jnp
from jax import lax
from jax.experimental import pallas as pl
from jax.experimental.pallas import tpu as pltpu
from jax.experimental.pallas import tpu_sc as plsc

_N = 100000
_NPAD = 102400
_CH = 10240
_NCHK = 5
_EDGES = 1600000
_EROWS = 12544
_E2 = _EROWS * 128
_RPT = _EROWS // 16
_BB = 8
_NBAT = _RPT // _BB
_WPT = _CH // 16
_ZR = 640
_NG = 64
_HID = 128
_NCG = _HID // 16
_BLK = 512
_GRID = _NPAD // _BLK


def _sc_mesh():
  return plsc.VectorSubcoreMesh(
      core_axis_name="c", subcore_axis_name="s", num_cores=2, num_subcores=16)


def _make_sc_pass(nplanes, with_gather):
  scratch = [
      pltpu.VMEM((_BB, 128), jnp.int32),
      pltpu.VMEM((_BB, 128), jnp.int32),
      pltpu.VMEM((_BB, 128, 16), jnp.float32),
      pltpu.VMEM((_ZR, 16), jnp.float32),
      pltpu.VMEM_SHARED((_CH, 16), jnp.float32),
      pltpu.SemaphoreType.DMA,
      pltpu.SemaphoreType.DMA,
  ]

  def body(*refs):
    if with_gather:
      srcm, dstm, tbl = refs[0], refs[1], refs[2]
      out = refs[3]
      dstb, srcb, rows, zbuf, acc, sem, sem2 = refs[4:]
    else:
      srcm, dstm = refs[0], refs[1]
      out = refs[2]
      dstb, srcb, rows, zbuf, acc, sem, sem2 = refs[3:]

    core = lax.axis_index("c")
    sub = lax.axis_index("s")
    row0 = sub * _RPT

    def zfill(i, _):
      zbuf[i, :] = jnp.zeros((16,), jnp.float32)
      return 0
    lax.fori_loop(0, _ZR, zfill, 0)

    if not with_gather:
      def ofill(i, _):
        t = i // 128
        r = i - t * 128
        rows[t, r, :] = jnp.ones((16,), jnp.float32)
        return 0
      lax.fori_loop(0, _BB * 128, ofill, 0)

    def plane_loop(p, _):
     def chunk_loop(k, _):
      cidx = core * _NCHK + k

      def zcp(i, _):
        pltpu.sync_copy(zbuf, acc.at[pl.ds(sub * _WPT + i * _ZR, _ZR)])
        return 0
      lax.fori_loop(0, _WPT // _ZR, zcp, 0)
      plsc.subcore_barrier()

      def blk(b, _):
        ebase = row0 + b * _BB
        pltpu.sync_copy(dstm.at[cidx, pl.ds(ebase, _BB)], dstb)
        if with_gather:
          pltpu.sync_copy(srcm.at[cidx, pl.ds(ebase, _BB)], srcb)

        if with_gather:
          gd = [pltpu.async_copy(
              tbl.at[p].at[plsc.Indices(srcb.at[t], ignored_value=-1)],
              rows.at[t], sem) for t in range(_BB)]
          for d in gd:
            d.wait()
        sd = [pltpu.async_copy(
            rows.at[t], acc.at[plsc.Indices(dstb.at[t], ignored_value=-1)],
            sem2, add=True) for t in range(_BB)]
        for d in sd:
          d.wait()
        return 0
      lax.fori_loop(0, _NBAT, blk, 0)
      plsc.subcore_barrier()

      def wb(i, _):
        pltpu.sync_copy(
            acc.at[pl.ds(sub * _WPT + i * _ZR, _ZR)],
            out.at[p, pl.ds(cidx * _CH + sub * _WPT + i * _ZR, _ZR)])
        return 0
      lax.fori_loop(0, _WPT // _ZR, wb, 0)
      plsc.subcore_barrier()
      return 0
     lax.fori_loop(0, _NCHK, chunk_loop, 0)
     return 0

    lax.fori_loop(0, nplanes, plane_loop, 0)

  return pl.kernel(
      body,
      out_type=pltpu.HBM((nplanes, _NPAD, 16), jnp.float32),
      mesh=_sc_mesh(),
      scratch_types=scratch,
      compiler_params=pltpu.CompilerParams(use_tc_tiling_on_sc=False),
  )


def _row_spec(shape_prefix):
  nd = len(shape_prefix)
  return pl.BlockSpec(shape_prefix + (_BLK, 16),
                      lambda i: (0,) * nd + (i, 0))


def _full_spec(shape):
  return pl.BlockSpec(shape, lambda i: (0,) * len(shape))


def _t0_body(deg_ref, xp_ref, dinv_ref, xt_ref):
  dinv = lax.rsqrt(1.0 + deg_ref[...])
  dinv_ref[...] = dinv
  xt_ref[...] = dinv * xp_ref[...]


def _t1_body(s1_ref, xt_ref, dinv_ref, w1_ref, b1_ref, w2_ref, out_ref):
  dinvc = dinv_ref[...][:, 0:1]
  agg = dinvc * (s1_ref[...] + xt_ref[...])
  h1 = jnp.maximum(
      jnp.dot(agg, w1_ref[...], preferred_element_type=jnp.float32)
      + b1_ref[...], 0.0)
  u2 = dinvc * jnp.dot(h1, w2_ref[...], preferred_element_type=jnp.float32)
  out_ref[...] = jnp.stack([u2[:, i * 16:(i + 1) * 16] for i in range(_NCG)])


def _t2_body(s_ref, u_ref, dinv_ref, b_ref, w_ref, out_ref):
  s = s_ref[...]
  u = u_ref[...]
  dinvc = dinv_ref[...][:, 0:1]
  b = b_ref[...]
  w = w_ref[...]
  u3 = None
  for p in range(_NCG):
    agg = dinvc * (s[p] + u[p])
    h2 = jnp.maximum(agg + b[:, p * 16:(p + 1) * 16], 0.0)
    part = jnp.dot(h2, w[p * 16:(p + 1) * 16, :],
                   preferred_element_type=jnp.float32)
    u3 = part if u3 is None else u3 + part
  u3 = dinvc * u3
  out_ref[...] = jnp.stack([u3[:, i * 16:(i + 1) * 16] for i in range(_NCG)])


def _t3_body(*refs):
  s_ref, u_ref, dinv_ref, b_ref, batch_ref = refs[:5]
  souts = refs[5:5 + _NCG]
  cnt_ref = refs[5 + _NCG]
  s = s_ref[...]
  u = u_ref[...]
  dinvc = dinv_ref[...][:, 0:1]
  b = b_ref[...]
  bv = batch_ref[...][:, 0]
  ids = lax.broadcasted_iota(jnp.int32, (_BLK, _NG), 1)
  oh = (bv[:, None] == ids).astype(jnp.float32)

  @pl.when(pl.program_id(0) == 0)
  def _init():
    for i in range(_NCG):
      souts[i][...] = jnp.zeros((_NG, 16), jnp.float32)
    cnt_ref[...] = jnp.zeros((_NG, 16), jnp.float32)

  for p in range(_NCG):
    h3 = jnp.maximum(dinvc * (s[p] + u[p])
                     + b[:, p * 16:(p + 1) * 16], 0.0)
    souts[p][...] += lax.dot_general(
        oh, h3, (((0,), (0,)), ((), ())), preferred_element_type=jnp.float32)
  c = jnp.sum(oh, axis=0)
  cnt_ref[...] += jnp.broadcast_to(c[:, None], (_NG, 16))


def _t4_body(*refs):
  sg = refs[0:_NCG]
  cnt_ref, wf1_ref, bf1_ref, wf2_ref, bf2_ref, out_ref = refs[_NCG:]
  inv_cnt = 1.0 / jnp.maximum(cnt_ref[...][:, 0:1], 1.0)
  wf1 = wf1_ref[...]
  acc = None
  for p in range(_NCG):
    g = sg[p][...] * inv_cnt
    part = jnp.dot(g, wf1[p * 16:(p + 1) * 16, :],
                   preferred_element_type=jnp.float32)
    acc = part if acc is None else acc + part
  fc1 = jnp.maximum(acc + bf1_ref[...], 0.0)
  out_ref[...] = (jnp.dot(fc1, wf2_ref[...],
                          preferred_element_type=jnp.float32) + bf2_ref[...])


def kernel(x, edge_index, batch, Wc1, bc1, Wc2, bc2, Wc3, bc3,
           Wf1, bf1, Wf2, bf2):
  f32 = jnp.float32
  npadextra = _NPAD - _N
  epadextra = _E2 - _EDGES

  pad_ids = jnp.arange(epadextra, dtype=jnp.int32)
  src_p = jnp.concatenate([edge_index[0], pad_ids % _N])
  dst_p = jnp.concatenate([edge_index[1], _N + (pad_ids % npadextra)])
  neg1 = jnp.int32(-1)
  dparts, sparts = [], []
  for chunk in range(2 * _NCHK):
    base = chunk * _CH
    inr = (dst_p >= base) & (dst_p < base + _CH)
    dparts.append(jnp.where(inr, dst_p - base, neg1))
    sparts.append(jnp.where(inr, src_p, neg1))
  dstm = jnp.stack(dparts).reshape(2 * _NCHK, _EROWS, 128)
  srcm = jnp.stack(sparts).reshape(2 * _NCHK, _EROWS, 128)

  xpad = jnp.pad(x, ((0, npadextra), (0, 16 - x.shape[1])))
  w1p = jnp.pad(Wc1, ((0, 16 - Wc1.shape[0]), (0, 0)))
  batchcol = jnp.pad(batch, (0, npadextra),
                     constant_values=_NG).reshape(_NPAD, 1)
  wf2p = jnp.pad(Wf2, ((0, 0), (0, 128 - Wf2.shape[1])))
  bf2p = jnp.pad(bf2, (0, 128 - bf2.shape[0])).reshape(1, 128)

  deg = _make_sc_pass(1, False)(srcm, dstm)[0]

  dinv16, xt = pl.pallas_call(
      _t0_body,
      grid=(_GRID,),
      in_specs=[_row_spec(()), _row_spec(())],
      out_specs=[_row_spec(()), _row_spec(())],
      out_shape=[jax.ShapeDtypeStruct((_NPAD, 16), f32)] * 2,
  )(deg, xpad)

  s1 = _make_sc_pass(1, True)(srcm, dstm, xt[None])[0]

  cg_shape = jax.ShapeDtypeStruct((_NCG, _NPAD, 16), f32)
  u2s = pl.pallas_call(
      _t1_body,
      grid=(_GRID,),
      in_specs=[_row_spec(()), _row_spec(()), _row_spec(()),
                _full_spec((16, _HID)), _full_spec((1, _HID)),
                _full_spec((_HID, _HID))],
      out_specs=_row_spec((_NCG,)),
      out_shape=cg_shape,
  )(s1, xt, dinv16, w1p, bc1.reshape(1, _HID), Wc2)

  s2 = _make_sc_pass(_NCG, True)(srcm, dstm, u2s)

  u3s = pl.pallas_call(
      _t2_body,
      grid=(_GRID,),
      in_specs=[_row_spec((_NCG,)), _row_spec((_NCG,)), _row_spec(()),
                _full_spec((1, _HID)), _full_spec((_HID, _HID))],
      out_specs=_row_spec((_NCG,)),
      out_shape=cg_shape,
  )(s2, u2s, dinv16, bc2.reshape(1, _HID), Wc3)

  s3 = _make_sc_pass(_NCG, True)(srcm, dstm, u3s)

  pool = pl.pallas_call(
      _t3_body,
      grid=(_GRID,),
      in_specs=[_row_spec((_NCG,)), _row_spec((_NCG,)), _row_spec(()),
                _full_spec((1, _HID)),
                pl.BlockSpec((_BLK, 1), lambda i: (i, 0))],
      out_specs=[_full_spec((_NG, 16))] * (_NCG + 1),
      out_shape=[jax.ShapeDtypeStruct((_NG, 16), f32)] * (_NCG + 1),
  )(s3, u3s, dinv16, bc3.reshape(1, _HID), batchcol)
  sumsg, cnt = pool[:_NCG], pool[_NCG]

  out128 = pl.pallas_call(
      _t4_body,
      grid=(1,),
      in_specs=[_full_spec((_NG, 16))] * (_NCG + 1)
      + [_full_spec((_HID, _NG)), _full_spec((1, _NG)),
         _full_spec((_NG, 128)), _full_spec((1, 128))],
      out_specs=_full_spec((_NG, 128)),
      out_shape=jax.ShapeDtypeStruct((_NG, 128), f32),
  )(*sumsg, cnt, Wf1, bf1.reshape(1, _NG), wf2p, bf2p)

  return out128[:, :1]


# speedup vs baseline: 2.4874x; 1.2578x over previous
import jax
import jax.numpy as jnp
from jax import lax
from jax.experimental import pallas as pl
from jax.experimental.pallas import tpu as pltpu
from jax.experimental.pallas import tpu_sc as plsc

_N = 100000
_NPAD = 102400
_CH = 10240
_NCHK = 5
_EDGES = 1600000
_EROWS = 12544
_E2 = _EROWS * 128
_RPT = _EROWS // 16
_BB = 16
_NBAT = _RPT // _BB
_WPT = _CH // 16
_ZR = 640
_NG = 64
_HID = 128
_NCG = _HID // 16
_BLK = 512
_GRID = _NPAD // _BLK


def _sc_mesh():
  return plsc.VectorSubcoreMesh(
      core_axis_name="c", subcore_axis_name="s", num_cores=2, num_subcores=16)


def _make_sc_pass(nplanes, with_gather):
  scratch = [
      pltpu.VMEM((_BB, 128), jnp.int32),
      pltpu.VMEM((_BB, 128), jnp.int32),
      pltpu.VMEM((_BB, 128, 16), jnp.float32),
      pltpu.VMEM((_ZR, 16), jnp.float32),
      pltpu.VMEM_SHARED((_CH, 16), jnp.float32),
      pltpu.SemaphoreType.DMA,
      pltpu.SemaphoreType.DMA,
  ]

  def body(*refs):
    if with_gather:
      srcm, dstm, tbl = refs[0], refs[1], refs[2]
      out = refs[3]
      dstb, srcb, rows, zbuf, acc, sem, sem2 = refs[4:]
    else:
      srcm, dstm = refs[0], refs[1]
      out = refs[2]
      dstb, srcb, rows, zbuf, acc, sem, sem2 = refs[3:]

    core = lax.axis_index("c")
    sub = lax.axis_index("s")
    row0 = sub * _RPT

    def zfill(i, _):
      zbuf[i, :] = jnp.zeros((16,), jnp.float32)
      return 0
    lax.fori_loop(0, _ZR, zfill, 0)

    if not with_gather:
      def ofill(i, _):
        t = i // 128
        r = i - t * 128
        rows[t, r, :] = jnp.ones((16,), jnp.float32)
        return 0
      lax.fori_loop(0, _BB * 128, ofill, 0)

    def plane_loop(p, _):
     def chunk_loop(k, _):
      cidx = core * _NCHK + k

      def zcp(i, _):
        pltpu.sync_copy(zbuf, acc.at[pl.ds(sub * _WPT + i * _ZR, _ZR)])
        return 0
      lax.fori_loop(0, _WPT // _ZR, zcp, 0)
      plsc.subcore_barrier()

      def blk(b, _):
        ebase = row0 + b * _BB
        pltpu.sync_copy(dstm.at[cidx, pl.ds(ebase, _BB)], dstb)
        if with_gather:
          pltpu.sync_copy(srcm.at[cidx, pl.ds(ebase, _BB)], srcb)

        if with_gather:
          gd = [pltpu.async_copy(
              tbl.at[p].at[plsc.Indices(srcb.at[t], ignored_value=-1)],
              rows.at[t], sem) for t in range(_BB)]
          for d in gd:
            d.wait()
        sd = [pltpu.async_copy(
            rows.at[t], acc.at[plsc.Indices(dstb.at[t], ignored_value=-1)],
            sem2, add=True) for t in range(_BB)]
        for d in sd:
          d.wait()
        return 0
      lax.fori_loop(0, _NBAT, blk, 0)
      plsc.subcore_barrier()

      def wb(i, _):
        pltpu.sync_copy(
            acc.at[pl.ds(sub * _WPT + i * _ZR, _ZR)],
            out.at[p, pl.ds(cidx * _CH + sub * _WPT + i * _ZR, _ZR)])
        return 0
      lax.fori_loop(0, _WPT // _ZR, wb, 0)
      plsc.subcore_barrier()
      return 0
     lax.fori_loop(0, _NCHK, chunk_loop, 0)
     return 0

    lax.fori_loop(0, nplanes, plane_loop, 0)

  return pl.kernel(
      body,
      out_type=pltpu.HBM((nplanes, _NPAD, 16), jnp.float32),
      mesh=_sc_mesh(),
      scratch_types=scratch,
      compiler_params=pltpu.CompilerParams(use_tc_tiling_on_sc=False),
  )


def _row_spec(shape_prefix):
  nd = len(shape_prefix)
  return pl.BlockSpec(shape_prefix + (_BLK, 16),
                      lambda i: (0,) * nd + (i, 0))


def _full_spec(shape):
  return pl.BlockSpec(shape, lambda i: (0,) * len(shape))


def _t0_body(deg_ref, xp_ref, dinv_ref, xt_ref):
  dinv = lax.rsqrt(1.0 + deg_ref[...])
  dinv_ref[...] = dinv
  xt_ref[...] = dinv * xp_ref[...]


def _t1_body(s1_ref, xt_ref, dinv_ref, w1_ref, b1_ref, w2_ref, out_ref):
  dinvc = dinv_ref[...][:, 0:1]
  agg = dinvc * (s1_ref[...] + xt_ref[...])
  h1 = jnp.maximum(
      jnp.dot(agg, w1_ref[...], preferred_element_type=jnp.float32)
      + b1_ref[...], 0.0)
  u2 = dinvc * jnp.dot(h1, w2_ref[...], preferred_element_type=jnp.float32)
  out_ref[...] = jnp.stack([u2[:, i * 16:(i + 1) * 16] for i in range(_NCG)])


def _t2_body(s_ref, u_ref, dinv_ref, b_ref, w_ref, out_ref):
  s = s_ref[...]
  u = u_ref[...]
  dinvc = dinv_ref[...][:, 0:1]
  b = b_ref[...]
  w = w_ref[...]
  u3 = None
  for p in range(_NCG):
    agg = dinvc * (s[p] + u[p])
    h2 = jnp.maximum(agg + b[:, p * 16:(p + 1) * 16], 0.0)
    part = jnp.dot(h2, w[p * 16:(p + 1) * 16, :],
                   preferred_element_type=jnp.float32)
    u3 = part if u3 is None else u3 + part
  u3 = dinvc * u3
  out_ref[...] = jnp.stack([u3[:, i * 16:(i + 1) * 16] for i in range(_NCG)])


def _t3_body(*refs):
  s_ref, u_ref, dinv_ref, b_ref, batch_ref = refs[:5]
  souts = refs[5:5 + _NCG]
  cnt_ref = refs[5 + _NCG]
  s = s_ref[...]
  u = u_ref[...]
  dinvc = dinv_ref[...][:, 0:1]
  b = b_ref[...]
  bv = batch_ref[...][:, 0]
  ids = lax.broadcasted_iota(jnp.int32, (_BLK, _NG), 1)
  oh = (bv[:, None] == ids).astype(jnp.float32)

  @pl.when(pl.program_id(0) == 0)
  def _init():
    for i in range(_NCG):
      souts[i][...] = jnp.zeros((_NG, 16), jnp.float32)
    cnt_ref[...] = jnp.zeros((_NG, 16), jnp.float32)

  for p in range(_NCG):
    h3 = jnp.maximum(dinvc * (s[p] + u[p])
                     + b[:, p * 16:(p + 1) * 16], 0.0)
    souts[p][...] += lax.dot_general(
        oh, h3, (((0,), (0,)), ((), ())), preferred_element_type=jnp.float32)
  c = jnp.sum(oh, axis=0)
  cnt_ref[...] += jnp.broadcast_to(c[:, None], (_NG, 16))


def _t4_body(*refs):
  sg = refs[0:_NCG]
  cnt_ref, wf1_ref, bf1_ref, wf2_ref, bf2_ref, out_ref = refs[_NCG:]
  inv_cnt = 1.0 / jnp.maximum(cnt_ref[...][:, 0:1], 1.0)
  wf1 = wf1_ref[...]
  acc = None
  for p in range(_NCG):
    g = sg[p][...] * inv_cnt
    part = jnp.dot(g, wf1[p * 16:(p + 1) * 16, :],
                   preferred_element_type=jnp.float32)
    acc = part if acc is None else acc + part
  fc1 = jnp.maximum(acc + bf1_ref[...], 0.0)
  out_ref[...] = (jnp.dot(fc1, wf2_ref[...],
                          preferred_element_type=jnp.float32) + bf2_ref[...])


def kernel(x, edge_index, batch, Wc1, bc1, Wc2, bc2, Wc3, bc3,
           Wf1, bf1, Wf2, bf2):
  f32 = jnp.float32
  npadextra = _NPAD - _N
  epadextra = _E2 - _EDGES

  pad_ids = jnp.arange(epadextra, dtype=jnp.int32)
  src_p = jnp.concatenate([edge_index[0], pad_ids % _N])
  dst_p = jnp.concatenate([edge_index[1], _N + (pad_ids % npadextra)])
  neg1 = jnp.int32(-1)
  dparts, sparts = [], []
  for chunk in range(2 * _NCHK):
    base = chunk * _CH
    inr = (dst_p >= base) & (dst_p < base + _CH)
    dparts.append(jnp.where(inr, dst_p - base, neg1))
    sparts.append(jnp.where(inr, src_p, neg1))
  dstm = jnp.stack(dparts).reshape(2 * _NCHK, _EROWS, 128)
  srcm = jnp.stack(sparts).reshape(2 * _NCHK, _EROWS, 128)

  xpad = jnp.pad(x, ((0, npadextra), (0, 16 - x.shape[1])))
  w1p = jnp.pad(Wc1, ((0, 16 - Wc1.shape[0]), (0, 0)))
  batchcol = jnp.pad(batch, (0, npadextra),
                     constant_values=_NG).reshape(_NPAD, 1)
  wf2p = jnp.pad(Wf2, ((0, 0), (0, 128 - Wf2.shape[1])))
  bf2p = jnp.pad(bf2, (0, 128 - bf2.shape[0])).reshape(1, 128)

  deg = _make_sc_pass(1, False)(srcm, dstm)[0]

  dinv16, xt = pl.pallas_call(
      _t0_body,
      grid=(_GRID,),
      in_specs=[_row_spec(()), _row_spec(())],
      out_specs=[_row_spec(()), _row_spec(())],
      out_shape=[jax.ShapeDtypeStruct((_NPAD, 16), f32)] * 2,
  )(deg, xpad)

  s1 = _make_sc_pass(1, True)(srcm, dstm, xt[None])[0]

  cg_shape = jax.ShapeDtypeStruct((_NCG, _NPAD, 16), f32)
  u2s = pl.pallas_call(
      _t1_body,
      grid=(_GRID,),
      in_specs=[_row_spec(()), _row_spec(()), _row_spec(()),
                _full_spec((16, _HID)), _full_spec((1, _HID)),
                _full_spec((_HID, _HID))],
      out_specs=_row_spec((_NCG,)),
      out_shape=cg_shape,
  )(s1, xt, dinv16, w1p, bc1.reshape(1, _HID), Wc2)

  s2 = _make_sc_pass(_NCG, True)(srcm, dstm, u2s)

  u3s = pl.pallas_call(
      _t2_body,
      grid=(_GRID,),
      in_specs=[_row_spec((_NCG,)), _row_spec((_NCG,)), _row_spec(()),
                _full_spec((1, _HID)), _full_spec((_HID, _HID))],
      out_specs=_row_spec((_NCG,)),
      out_shape=cg_shape,
  )(s2, u2s, dinv16, bc2.reshape(1, _HID), Wc3)

  s3 = _make_sc_pass(_NCG, True)(srcm, dstm, u3s)

  pool = pl.pallas_call(
      _t3_body,
      grid=(_GRID,),
      in_specs=[_row_spec((_NCG,)), _row_spec((_NCG,)), _row_spec(()),
                _full_spec((1, _HID)),
                pl.BlockSpec((_BLK, 1), lambda i: (i, 0))],
      out_specs=[_full_spec((_NG, 16))] * (_NCG + 1),
      out_shape=[jax.ShapeDtypeStruct((_NG, 16), f32)] * (_NCG + 1),
  )(s3, u3s, dinv16, bc3.reshape(1, _HID), batchcol)
  sumsg, cnt = pool[:_NCG], pool[_NCG]

  out128 = pl.pallas_call(
      _t4_body,
      grid=(1,),
      in_specs=[_full_spec((_NG, 16))] * (_NCG + 1)
      + [_full_spec((_HID, _NG)), _full_spec((1, _NG)),
         _full_spec((_NG, 128)), _full_spec((1, 128))],
      out_specs=_full_spec((_NG, 128)),
      out_shape=jax.ShapeDtypeStruct((_NG, 128), f32),
  )(*sumsg, cnt, Wf1, bf1.reshape(1, _NG), wf2p, bf2p)

  return out128[:, :1]


# speedup vs baseline: 2.8203x; 1.1338x over previous
import jax
import jax.numpy as jnp
from jax import lax
from jax.experimental import pallas as pl
from jax.experimental.pallas import tpu as pltpu
from jax.experimental.pallas import tpu_sc as plsc

_N = 100000
_NPAD = 102400
_CH = 10240
_NCHK = 5
_EDGES = 1600000
_EROWS = 12544
_E2 = _EROWS * 128
_RPT = _EROWS // 16
_BB = 28
_NBAT = _RPT // _BB
_WPT = _CH // 16
_ZR = 640
_NG = 64
_HID = 128
_NCG = _HID // 16
_BLK = 512
_GRID = _NPAD // _BLK


def _sc_mesh():
  return plsc.VectorSubcoreMesh(
      core_axis_name="c", subcore_axis_name="s", num_cores=2, num_subcores=16)


def _make_sc_pass(nplanes, with_gather):
  scratch = [
      pltpu.VMEM((_BB, 128), jnp.int32),
      pltpu.VMEM((_BB, 128), jnp.int32),
      pltpu.VMEM((_BB, 128, 16), jnp.float32),
      pltpu.VMEM((_ZR, 16), jnp.float32),
      pltpu.VMEM_SHARED((_CH, 16), jnp.float32),
      pltpu.SemaphoreType.DMA,
      pltpu.SemaphoreType.DMA,
  ]

  def body(*refs):
    if with_gather:
      srcm, dstm, tbl = refs[0], refs[1], refs[2]
      out = refs[3]
      dstb, srcb, rows, zbuf, acc, sem, sem2 = refs[4:]
    else:
      srcm, dstm = refs[0], refs[1]
      out = refs[2]
      dstb, srcb, rows, zbuf, acc, sem, sem2 = refs[3:]

    core = lax.axis_index("c")
    sub = lax.axis_index("s")
    row0 = sub * _RPT

    def zfill(i, _):
      zbuf[i, :] = jnp.zeros((16,), jnp.float32)
      return 0
    lax.fori_loop(0, _ZR, zfill, 0)

    if not with_gather:
      def ofill(i, _):
        t = i // 128
        r = i - t * 128
        rows[t, r, :] = jnp.ones((16,), jnp.float32)
        return 0
      lax.fori_loop(0, _BB * 128, ofill, 0)

    def plane_loop(p, _):
     def chunk_loop(k, _):
      cidx = core * _NCHK + k

      def zcp(i, _):
        pltpu.sync_copy(zbuf, acc.at[pl.ds(sub * _WPT + i * _ZR, _ZR)])
        return 0
      lax.fori_loop(0, _WPT // _ZR, zcp, 0)
      plsc.subcore_barrier()

      def blk(b, _):
        ebase = row0 + b * _BB
        pltpu.sync_copy(dstm.at[cidx, pl.ds(ebase, _BB)], dstb)
        if with_gather:
          pltpu.sync_copy(srcm.at[cidx, pl.ds(ebase, _BB)], srcb)

        if with_gather:
          gd = [pltpu.async_copy(
              tbl.at[p].at[plsc.Indices(srcb.at[t], ignored_value=-1)],
              rows.at[t], sem) for t in range(_BB)]
          for d in gd:
            d.wait()
        sd = [pltpu.async_copy(
            rows.at[t], acc.at[plsc.Indices(dstb.at[t], ignored_value=-1)],
            sem2, add=True) for t in range(_BB)]
        for d in sd:
          d.wait()
        return 0
      lax.fori_loop(0, _NBAT, blk, 0)
      plsc.subcore_barrier()

      def wb(i, _):
        pltpu.sync_copy(
            acc.at[pl.ds(sub * _WPT + i * _ZR, _ZR)],
            out.at[p, pl.ds(cidx * _CH + sub * _WPT + i * _ZR, _ZR)])
        return 0
      lax.fori_loop(0, _WPT // _ZR, wb, 0)
      plsc.subcore_barrier()
      return 0
     lax.fori_loop(0, _NCHK, chunk_loop, 0)
     return 0

    lax.fori_loop(0, nplanes, plane_loop, 0)

  return pl.kernel(
      body,
      out_type=pltpu.HBM((nplanes, _NPAD, 16), jnp.float32),
      mesh=_sc_mesh(),
      scratch_types=scratch,
      compiler_params=pltpu.CompilerParams(use_tc_tiling_on_sc=False),
  )


def _row_spec(shape_prefix):
  nd = len(shape_prefix)
  return pl.BlockSpec(shape_prefix + (_BLK, 16),
                      lambda i: (0,) * nd + (i, 0))


def _full_spec(shape):
  return pl.BlockSpec(shape, lambda i: (0,) * len(shape))


def _t0_body(deg_ref, xp_ref, dinv_ref, xt_ref):
  dinv = lax.rsqrt(1.0 + deg_ref[...])
  dinv_ref[...] = dinv
  xt_ref[...] = dinv * xp_ref[...]


def _t1_body(s1_ref, xt_ref, dinv_ref, w1_ref, b1_ref, w2_ref, out_ref):
  dinvc = dinv_ref[...][:, 0:1]
  agg = dinvc * (s1_ref[...] + xt_ref[...])
  h1 = jnp.maximum(
      jnp.dot(agg, w1_ref[...], preferred_element_type=jnp.float32)
      + b1_ref[...], 0.0)
  u2 = dinvc * jnp.dot(h1, w2_ref[...], preferred_element_type=jnp.float32)
  out_ref[...] = jnp.stack([u2[:, i * 16:(i + 1) * 16] for i in range(_NCG)])


def _t2_body(s_ref, u_ref, dinv_ref, b_ref, w_ref, out_ref):
  s = s_ref[...]
  u = u_ref[...]
  dinvc = dinv_ref[...][:, 0:1]
  b = b_ref[...]
  w = w_ref[...]
  u3 = None
  for p in range(_NCG):
    agg = dinvc * (s[p] + u[p])
    h2 = jnp.maximum(agg + b[:, p * 16:(p + 1) * 16], 0.0)
    part = jnp.dot(h2, w[p * 16:(p + 1) * 16, :],
                   preferred_element_type=jnp.float32)
    u3 = part if u3 is None else u3 + part
  u3 = dinvc * u3
  out_ref[...] = jnp.stack([u3[:, i * 16:(i + 1) * 16] for i in range(_NCG)])


def _t3_body(*refs):
  s_ref, u_ref, dinv_ref, b_ref, batch_ref = refs[:5]
  souts = refs[5:5 + _NCG]
  cnt_ref = refs[5 + _NCG]
  s = s_ref[...]
  u = u_ref[...]
  dinvc = dinv_ref[...][:, 0:1]
  b = b_ref[...]
  bv = batch_ref[...][:, 0]
  ids = lax.broadcasted_iota(jnp.int32, (_BLK, _NG), 1)
  oh = (bv[:, None] == ids).astype(jnp.float32)

  @pl.when(pl.program_id(0) == 0)
  def _init():
    for i in range(_NCG):
      souts[i][...] = jnp.zeros((_NG, 16), jnp.float32)
    cnt_ref[...] = jnp.zeros((_NG, 16), jnp.float32)

  for p in range(_NCG):
    h3 = jnp.maximum(dinvc * (s[p] + u[p])
                     + b[:, p * 16:(p + 1) * 16], 0.0)
    souts[p][...] += lax.dot_general(
        oh, h3, (((0,), (0,)), ((), ())), preferred_element_type=jnp.float32)
  c = jnp.sum(oh, axis=0)
  cnt_ref[...] += jnp.broadcast_to(c[:, None], (_NG, 16))


def _t4_body(*refs):
  sg = refs[0:_NCG]
  cnt_ref, wf1_ref, bf1_ref, wf2_ref, bf2_ref, out_ref = refs[_NCG:]
  inv_cnt = 1.0 / jnp.maximum(cnt_ref[...][:, 0:1], 1.0)
  wf1 = wf1_ref[...]
  acc = None
  for p in range(_NCG):
    g = sg[p][...] * inv_cnt
    part = jnp.dot(g, wf1[p * 16:(p + 1) * 16, :],
                   preferred_element_type=jnp.float32)
    acc = part if acc is None else acc + part
  fc1 = jnp.maximum(acc + bf1_ref[...], 0.0)
  out_ref[...] = (jnp.dot(fc1, wf2_ref[...],
                          preferred_element_type=jnp.float32) + bf2_ref[...])


def kernel(x, edge_index, batch, Wc1, bc1, Wc2, bc2, Wc3, bc3,
           Wf1, bf1, Wf2, bf2):
  f32 = jnp.float32
  npadextra = _NPAD - _N
  epadextra = _E2 - _EDGES

  pad_ids = jnp.arange(epadextra, dtype=jnp.int32)
  src_p = jnp.concatenate([edge_index[0], pad_ids % _N])
  dst_p = jnp.concatenate([edge_index[1], _N + (pad_ids % npadextra)])
  neg1 = jnp.int32(-1)
  dparts, sparts = [], []
  for chunk in range(2 * _NCHK):
    base = chunk * _CH
    inr = (dst_p >= base) & (dst_p < base + _CH)
    dparts.append(jnp.where(inr, dst_p - base, neg1))
    sparts.append(jnp.where(inr, src_p, neg1))
  dstm = jnp.stack(dparts).reshape(2 * _NCHK, _EROWS, 128)
  srcm = jnp.stack(sparts).reshape(2 * _NCHK, _EROWS, 128)

  xpad = jnp.pad(x, ((0, npadextra), (0, 16 - x.shape[1])))
  w1p = jnp.pad(Wc1, ((0, 16 - Wc1.shape[0]), (0, 0)))
  batchcol = jnp.pad(batch, (0, npadextra),
                     constant_values=_NG).reshape(_NPAD, 1)
  wf2p = jnp.pad(Wf2, ((0, 0), (0, 128 - Wf2.shape[1])))
  bf2p = jnp.pad(bf2, (0, 128 - bf2.shape[0])).reshape(1, 128)

  deg = _make_sc_pass(1, False)(srcm, dstm)[0]

  dinv16, xt = pl.pallas_call(
      _t0_body,
      grid=(_GRID,),
      in_specs=[_row_spec(()), _row_spec(())],
      out_specs=[_row_spec(()), _row_spec(())],
      out_shape=[jax.ShapeDtypeStruct((_NPAD, 16), f32)] * 2,
  )(deg, xpad)

  s1 = _make_sc_pass(1, True)(srcm, dstm, xt[None])[0]

  cg_shape = jax.ShapeDtypeStruct((_NCG, _NPAD, 16), f32)
  u2s = pl.pallas_call(
      _t1_body,
      grid=(_GRID,),
      in_specs=[_row_spec(()), _row_spec(()), _row_spec(()),
                _full_spec((16, _HID)), _full_spec((1, _HID)),
                _full_spec((_HID, _HID))],
      out_specs=_row_spec((_NCG,)),
      out_shape=cg_shape,
  )(s1, xt, dinv16, w1p, bc1.reshape(1, _HID), Wc2)

  s2 = _make_sc_pass(_NCG, True)(srcm, dstm, u2s)

  u3s = pl.pallas_call(
      _t2_body,
      grid=(_GRID,),
      in_specs=[_row_spec((_NCG,)), _row_spec((_NCG,)), _row_spec(()),
                _full_spec((1, _HID)), _full_spec((_HID, _HID))],
      out_specs=_row_spec((_NCG,)),
      out_shape=cg_shape,
  )(s2, u2s, dinv16, bc2.reshape(1, _HID), Wc3)

  s3 = _make_sc_pass(_NCG, True)(srcm, dstm, u3s)

  pool = pl.pallas_call(
      _t3_body,
      grid=(_GRID,),
      in_specs=[_row_spec((_NCG,)), _row_spec((_NCG,)), _row_spec(()),
                _full_spec((1, _HID)),
                pl.BlockSpec((_BLK, 1), lambda i: (i, 0))],
      out_specs=[_full_spec((_NG, 16))] * (_NCG + 1),
      out_shape=[jax.ShapeDtypeStruct((_NG, 16), f32)] * (_NCG + 1),
  )(s3, u3s, dinv16, bc3.reshape(1, _HID), batchcol)
  sumsg, cnt = pool[:_NCG], pool[_NCG]

  out128 = pl.pallas_call(
      _t4_body,
      grid=(1,),
      in_specs=[_full_spec((_NG, 16))] * (_NCG + 1)
      + [_full_spec((_HID, _NG)), _full_spec((1, _NG)),
         _full_spec((_NG, 128)), _full_spec((1, 128))],
      out_specs=_full_spec((_NG, 128)),
      out_shape=jax.ShapeDtypeStruct((_NG, 128), f32),
  )(*sumsg, cnt, Wf1, bf1.reshape(1, _NG), wf2p, bf2p)

  return out128[:, :1]


# speedup vs baseline: 3.5292x; 1.2514x over previous
import jax
import jax.numpy as jnp
from jax import lax
from jax.experimental import pallas as pl
from jax.experimental.pallas import tpu as pltpu
from jax.experimental.pallas import tpu_sc as plsc

_N = 100000
_NPAD = 102400
_CH = 10240
_NCHK = 5
_EDGES = 1600000
_EROWS = 12544
_E2 = _EROWS * 128
_RPT = _EROWS // 16
_BB = 28
_NBAT = _RPT // _BB
_WPT = _CH // 16
_ZR = 640
_NG = 64
_HID = 128
_NCG = _HID // 16
_BLK = 512
_GRID = _NPAD // _BLK


def _sc_mesh():
  return plsc.VectorSubcoreMesh(
      core_axis_name="c", subcore_axis_name="s", num_cores=2, num_subcores=16)


def _make_sc_pass(nplanes, with_gather):
  scratch = [
      pltpu.VMEM((_BB, 128), jnp.int32),
      pltpu.VMEM((_BB, 128), jnp.int32),
      pltpu.VMEM((_BB, 128, 16), jnp.float32),
      pltpu.VMEM((_ZR, 16), jnp.float32),
      pltpu.VMEM_SHARED((_CH, 16), jnp.float32),
      pltpu.SemaphoreType.DMA,
      pltpu.SemaphoreType.DMA,
  ]

  def body(*refs):
    if with_gather:
      srcm, dstm, tbl = refs[0], refs[1], refs[2]
      out = refs[3]
      dstb, srcb, rows, zbuf, acc, sem, sem2 = refs[4:]
    else:
      srcm, dstm = refs[0], refs[1]
      out = refs[2]
      dstb, srcb, rows, zbuf, acc, sem, sem2 = refs[3:]

    core = lax.axis_index("c")
    sub = lax.axis_index("s")
    row0 = sub * _RPT

    def zfill(i, _):
      zbuf[i, :] = jnp.zeros((16,), jnp.float32)
      return 0
    lax.fori_loop(0, _ZR, zfill, 0)

    if not with_gather:
      def ofill(i, _):
        t = i // 128
        r = i - t * 128
        rows[t, r, :] = jnp.ones((16,), jnp.float32)
        return 0
      lax.fori_loop(0, _BB * 128, ofill, 0)

    def plane_loop(p, _):
     def chunk_loop(k, _):
      cidx = core * _NCHK + k

      def zcp(i, _):
        pltpu.sync_copy(zbuf, acc.at[pl.ds(sub * _WPT + i * _ZR, _ZR)])
        return 0
      lax.fori_loop(0, _WPT // _ZR, zcp, 0)
      plsc.subcore_barrier()

      def blk(b, _):
        ebase = row0 + b * _BB
        pltpu.sync_copy(dstm.at[cidx, pl.ds(ebase, _BB)], dstb)
        if with_gather:
          pltpu.sync_copy(srcm.at[cidx, pl.ds(ebase, _BB)], srcb)

        if with_gather:
          gd = [pltpu.async_copy(
              tbl.at[p].at[plsc.Indices(srcb.at[t], ignored_value=-1)],
              rows.at[t], sem) for t in range(_BB)]
          sd = []
          for t in range(_BB):
            gd[t].wait()
            sd.append(pltpu.async_copy(
                rows.at[t],
                acc.at[plsc.Indices(dstb.at[t], ignored_value=-1)],
                sem2, add=True))
        else:
          sd = [pltpu.async_copy(
              rows.at[t],
              acc.at[plsc.Indices(dstb.at[t], ignored_value=-1)],
              sem2, add=True) for t in range(_BB)]
        for d in sd:
          d.wait()
        return 0
      lax.fori_loop(0, _NBAT, blk, 0)
      plsc.subcore_barrier()

      def wb(i, _):
        pltpu.sync_copy(
            acc.at[pl.ds(sub * _WPT + i * _ZR, _ZR)],
            out.at[p, pl.ds(cidx * _CH + sub * _WPT + i * _ZR, _ZR)])
        return 0
      lax.fori_loop(0, _WPT // _ZR, wb, 0)
      plsc.subcore_barrier()
      return 0
     lax.fori_loop(0, _NCHK, chunk_loop, 0)
     return 0

    lax.fori_loop(0, nplanes, plane_loop, 0)

  return pl.kernel(
      body,
      out_type=pltpu.HBM((nplanes, _NPAD, 16), jnp.float32),
      mesh=_sc_mesh(),
      scratch_types=scratch,
      compiler_params=pltpu.CompilerParams(use_tc_tiling_on_sc=False),
  )


def _row_spec(shape_prefix):
  nd = len(shape_prefix)
  return pl.BlockSpec(shape_prefix + (_BLK, 16),
                      lambda i: (0,) * nd + (i, 0))


def _full_spec(shape):
  return pl.BlockSpec(shape, lambda i: (0,) * len(shape))


def _t0_body(deg_ref, xp_ref, dinv_ref, xt_ref):
  dinv = lax.rsqrt(1.0 + deg_ref[...])
  dinv_ref[...] = dinv
  xt_ref[...] = dinv * xp_ref[...]


def _t1_body(s1_ref, xt_ref, dinv_ref, w1_ref, b1_ref, w2_ref, out_ref):
  dinvc = dinv_ref[...][:, 0:1]
  agg = dinvc * (s1_ref[...] + xt_ref[...])
  h1 = jnp.maximum(
      jnp.dot(agg, w1_ref[...], preferred_element_type=jnp.float32)
      + b1_ref[...], 0.0)
  u2 = dinvc * jnp.dot(h1, w2_ref[...], preferred_element_type=jnp.float32)
  out_ref[...] = jnp.stack([u2[:, i * 16:(i + 1) * 16] for i in range(_NCG)])


def _t2_body(s_ref, u_ref, dinv_ref, b_ref, w_ref, out_ref):
  s = s_ref[...]
  u = u_ref[...]
  dinvc = dinv_ref[...][:, 0:1]
  b = b_ref[...]
  w = w_ref[...]
  u3 = None
  for p in range(_NCG):
    agg = dinvc * (s[p] + u[p])
    h2 = jnp.maximum(agg + b[:, p * 16:(p + 1) * 16], 0.0)
    part = jnp.dot(h2, w[p * 16:(p + 1) * 16, :],
                   preferred_element_type=jnp.float32)
    u3 = part if u3 is None else u3 + part
  u3 = dinvc * u3
  out_ref[...] = jnp.stack([u3[:, i * 16:(i + 1) * 16] for i in range(_NCG)])


def _t3_body(*refs):
  s_ref, u_ref, dinv_ref, b_ref, batch_ref = refs[:5]
  souts = refs[5:5 + _NCG]
  cnt_ref = refs[5 + _NCG]
  s = s_ref[...]
  u = u_ref[...]
  dinvc = dinv_ref[...][:, 0:1]
  b = b_ref[...]
  bv = batch_ref[...][:, 0]
  ids = lax.broadcasted_iota(jnp.int32, (_BLK, _NG), 1)
  oh = (bv[:, None] == ids).astype(jnp.float32)

  @pl.when(pl.program_id(0) == 0)
  def _init():
    for i in range(_NCG):
      souts[i][...] = jnp.zeros((_NG, 16), jnp.float32)
    cnt_ref[...] = jnp.zeros((_NG, 16), jnp.float32)

  for p in range(_NCG):
    h3 = jnp.maximum(dinvc * (s[p] + u[p])
                     + b[:, p * 16:(p + 1) * 16], 0.0)
    souts[p][...] += lax.dot_general(
        oh, h3, (((0,), (0,)), ((), ())), preferred_element_type=jnp.float32)
  c = jnp.sum(oh, axis=0)
  cnt_ref[...] += jnp.broadcast_to(c[:, None], (_NG, 16))


def _t4_body(*refs):
  sg = refs[0:_NCG]
  cnt_ref, wf1_ref, bf1_ref, wf2_ref, bf2_ref, out_ref = refs[_NCG:]
  inv_cnt = 1.0 / jnp.maximum(cnt_ref[...][:, 0:1], 1.0)
  wf1 = wf1_ref[...]
  acc = None
  for p in range(_NCG):
    g = sg[p][...] * inv_cnt
    part = jnp.dot(g, wf1[p * 16:(p + 1) * 16, :],
                   preferred_element_type=jnp.float32)
    acc = part if acc is None else acc + part
  fc1 = jnp.maximum(acc + bf1_ref[...], 0.0)
  out_ref[...] = (jnp.dot(fc1, wf2_ref[...],
                          preferred_element_type=jnp.float32) + bf2_ref[...])


def kernel(x, edge_index, batch, Wc1, bc1, Wc2, bc2, Wc3, bc3,
           Wf1, bf1, Wf2, bf2):
  f32 = jnp.float32
  npadextra = _NPAD - _N
  epadextra = _E2 - _EDGES

  pad_ids = jnp.arange(epadextra, dtype=jnp.int32)
  src_p = jnp.concatenate([edge_index[0], pad_ids % _N])
  dst_p = jnp.concatenate([edge_index[1], _N + (pad_ids % npadextra)])
  neg1 = jnp.int32(-1)
  dparts, sparts = [], []
  for chunk in range(2 * _NCHK):
    base = chunk * _CH
    inr = (dst_p >= base) & (dst_p < base + _CH)
    dparts.append(jnp.where(inr, dst_p - base, neg1))
    sparts.append(jnp.where(inr, src_p, neg1))
  dstm = jnp.stack(dparts).reshape(2 * _NCHK, _EROWS, 128)
  srcm = jnp.stack(sparts).reshape(2 * _NCHK, _EROWS, 128)

  xpad = jnp.pad(x, ((0, npadextra), (0, 16 - x.shape[1])))
  w1p = jnp.pad(Wc1, ((0, 16 - Wc1.shape[0]), (0, 0)))
  batchcol = jnp.pad(batch, (0, npadextra),
                     constant_values=_NG).reshape(_NPAD, 1)
  wf2p = jnp.pad(Wf2, ((0, 0), (0, 128 - Wf2.shape[1])))
  bf2p = jnp.pad(bf2, (0, 128 - bf2.shape[0])).reshape(1, 128)

  deg = _make_sc_pass(1, False)(srcm, dstm)[0]

  dinv16, xt = pl.pallas_call(
      _t0_body,
      grid=(_GRID,),
      in_specs=[_row_spec(()), _row_spec(())],
      out_specs=[_row_spec(()), _row_spec(())],
      out_shape=[jax.ShapeDtypeStruct((_NPAD, 16), f32)] * 2,
  )(deg, xpad)

  s1 = _make_sc_pass(1, True)(srcm, dstm, xt[None])[0]

  cg_shape = jax.ShapeDtypeStruct((_NCG, _NPAD, 16), f32)
  u2s = pl.pallas_call(
      _t1_body,
      grid=(_GRID,),
      in_specs=[_row_spec(()), _row_spec(()), _row_spec(()),
                _full_spec((16, _HID)), _full_spec((1, _HID)),
                _full_spec((_HID, _HID))],
      out_specs=_row_spec((_NCG,)),
      out_shape=cg_shape,
  )(s1, xt, dinv16, w1p, bc1.reshape(1, _HID), Wc2)

  s2 = _make_sc_pass(_NCG, True)(srcm, dstm, u2s)

  u3s = pl.pallas_call(
      _t2_body,
      grid=(_GRID,),
      in_specs=[_row_spec((_NCG,)), _row_spec((_NCG,)), _row_spec(()),
                _full_spec((1, _HID)), _full_spec((_HID, _HID))],
      out_specs=_row_spec((_NCG,)),
      out_shape=cg_shape,
  )(s2, u2s, dinv16, bc2.reshape(1, _HID), Wc3)

  s3 = _make_sc_pass(_NCG, True)(srcm, dstm, u3s)

  pool = pl.pallas_call(
      _t3_body,
      grid=(_GRID,),
      in_specs=[_row_spec((_NCG,)), _row_spec((_NCG,)), _row_spec(()),
                _full_spec((1, _HID)),
                pl.BlockSpec((_BLK, 1), lambda i: (i, 0))],
      out_specs=[_full_spec((_NG, 16))] * (_NCG + 1),
      out_shape=[jax.ShapeDtypeStruct((_NG, 16), f32)] * (_NCG + 1),
  )(s3, u3s, dinv16, bc3.reshape(1, _HID), batchcol)
  sumsg, cnt = pool[:_NCG], pool[_NCG]

  out128 = pl.pallas_call(
      _t4_body,
      grid=(1,),
      in_specs=[_full_spec((_NG, 16))] * (_NCG + 1)
      + [_full_spec((_HID, _NG)), _full_spec((1, _NG)),
         _full_spec((_NG, 128)), _full_spec((1, 128))],
      out_specs=_full_spec((_NG, 128)),
      out_shape=jax.ShapeDtypeStruct((_NG, 128), f32),
  )(*sumsg, cnt, Wf1, bf1.reshape(1, _NG), wf2p, bf2p)

  return out128[:, :1]
